# initial kernel scaffold (unmeasured)
import jax
import jax.numpy as jnp
from jax import lax
from jax.experimental import pallas as pl
from jax.experimental.pallas import tpu as pltpu


def kernel(
    x,
):
    def body(*refs):
        pass

    out_shape = jax.ShapeDtypeStruct(..., jnp.float32)
    return pl.pallas_call(body, out_shape=out_shape)(...)


# baseline (device time: 32816 ns/iter reference)
import jax
import jax.numpy as jnp
from jax import lax
from jax.experimental import pallas as pl
from jax.experimental.pallas import tpu as pltpu


def kernel(x):
    m_per, n = x.shape
    half = m_per // 2

    def body(x_ref, out_ref, send_sems, recv_sems):
        my_x = lax.axis_index("x")
        my_y = lax.axis_index("y")
        x_nbr = (1 - my_x, my_y)
        y_nbr = (my_x, 1 - my_y)

        barrier_sem = pltpu.get_barrier_semaphore()
        for nbr in (x_nbr, y_nbr):
            pl.semaphore_signal(
                barrier_sem, inc=1,
                device_id=nbr, device_id_type=pl.DeviceIdType.MESH,
            )
        pl.semaphore_wait(barrier_sem, 2)

        out_ref[pl.ds(my_x * m_per, m_per), :] = x_ref[:, :].astype(out_ref.dtype)

        send_off = my_x * m_per + my_y * half
        rdma1 = pltpu.make_async_remote_copy(
            src_ref=out_ref.at[pl.ds(send_off, half), :],
            dst_ref=out_ref.at[pl.ds(send_off, half), :],
            send_sem=send_sems.at[0],
            recv_sem=recv_sems.at[0],
            device_id=x_nbr,
            device_id_type=pl.DeviceIdType.MESH,
        )
        rdma1.start()
        rdma1.wait()

        fwd_off = (1 - my_x) * m_per + my_y * half
        rdma2 = pltpu.make_async_remote_copy(
            src_ref=out_ref.at[pl.ds(fwd_off, half), :],
            dst_ref=out_ref.at[pl.ds(fwd_off, half), :],
            send_sem=send_sems.at[1],
            recv_sem=recv_sems.at[1],
            device_id=y_nbr,
            device_id_type=pl.DeviceIdType.MESH,
        )
        rdma2.start()
        rdma2.wait()

    return pl.pallas_call(
        body,
        out_shape=jax.ShapeDtypeStruct((2 * m_per, n), jnp.bfloat16),
        in_specs=[pl.BlockSpec(memory_space=pltpu.VMEM)],
        out_specs=pl.BlockSpec(memory_space=pltpu.VMEM),
        scratch_shapes=[
            pltpu.SemaphoreType.DMA((2,)),
            pltpu.SemaphoreType.DMA((2,)),
        ],
        compiler_params=pltpu.CompilerParams(collective_id=0),
    )(x)


# device time: 23181 ns/iter; 1.4156x vs baseline; 1.4156x over previous
import jax
import jax.numpy as jnp
from jax import lax
from jax.experimental import pallas as pl
from jax.experimental.pallas import tpu as pltpu


N_CHUNKS = 8


def kernel(x):
    m_per, n = x.shape
    half = m_per // 2
    chunk = half // N_CHUNKS

    def body(x_ref, out_ref, p1_send, p1_recv, p2_send, p2_recv):
        my_x = lax.axis_index("x")
        my_y = lax.axis_index("y")
        x_nbr = (1 - my_x, my_y)
        y_nbr = (my_x, 1 - my_y)

        barrier_sem = pltpu.get_barrier_semaphore()
        for nbr in (x_nbr, y_nbr):
            pl.semaphore_signal(
                barrier_sem, inc=1,
                device_id=nbr, device_id_type=pl.DeviceIdType.MESH,
            )
        pl.semaphore_wait(barrier_sem, 2)

        send_off = my_x * m_per + my_y * half
        keep_off = my_x * m_per + (1 - my_y) * half
        out_ref[pl.ds(send_off, half), :] = (
            x_ref[pl.ds(my_y * half, half), :].astype(out_ref.dtype)
        )

        p1 = []
        for c in range(N_CHUNKS):
            off = send_off + c * chunk
            rdma = pltpu.make_async_remote_copy(
                src_ref=out_ref.at[pl.ds(off, chunk), :],
                dst_ref=out_ref.at[pl.ds(off, chunk), :],
                send_sem=p1_send.at[c],
                recv_sem=p1_recv.at[c],
                device_id=x_nbr,
                device_id_type=pl.DeviceIdType.MESH,
            )
            rdma.start()
            p1.append(rdma)

        out_ref[pl.ds(keep_off, half), :] = (
            x_ref[pl.ds((1 - my_y) * half, half), :].astype(out_ref.dtype)
        )

        fwd_off = (1 - my_x) * m_per + my_y * half
        p2 = []
        for c in range(N_CHUNKS):
            p1[c].wait_recv()
            off = fwd_off + c * chunk
            rdma = pltpu.make_async_remote_copy(
                src_ref=out_ref.at[pl.ds(off, chunk), :],
                dst_ref=out_ref.at[pl.ds(off, chunk), :],
                send_sem=p2_send.at[c],
                recv_sem=p2_recv.at[c],
                device_id=y_nbr,
                device_id_type=pl.DeviceIdType.MESH,
            )
            rdma.start()
            p2.append(rdma)

        for c in range(N_CHUNKS):
            p1[c].wait_send()
            p2[c].wait()

    return pl.pallas_call(
        body,
        out_shape=jax.ShapeDtypeStruct((2 * m_per, n), jnp.bfloat16),
        in_specs=[pl.BlockSpec(memory_space=pltpu.VMEM)],
        out_specs=pl.BlockSpec(memory_space=pltpu.VMEM),
        scratch_shapes=[
            pltpu.SemaphoreType.DMA((N_CHUNKS,)),
            pltpu.SemaphoreType.DMA((N_CHUNKS,)),
            pltpu.SemaphoreType.DMA((N_CHUNKS,)),
            pltpu.SemaphoreType.DMA((N_CHUNKS,)),
        ],
        compiler_params=pltpu.CompilerParams(collective_id=0),
    )(x)


# device time: 22894 ns/iter; 1.4334x vs baseline; 1.0125x over previous
import jax
import jax.numpy as jnp
from jax import lax
from jax.experimental import pallas as pl
from jax.experimental.pallas import tpu as pltpu

N_CHUNKS = 16


def kernel(x):
    m_per, n = x.shape
    half = m_per // 2
    chunk = half // N_CHUNKS

    def body(x_ref, out_ref, sall, rbuf,
             p1_send, p1_recv, p2_send, p2_recv, dsem, dsem_own):
        my_x = lax.axis_index("x")
        my_y = lax.axis_index("y")
        x_nbr = (1 - my_x, my_y)
        y_nbr = (my_x, 1 - my_y)

        barrier_sem = pltpu.get_barrier_semaphore()
        for nbr in (x_nbr, y_nbr):
            pl.semaphore_signal(
                barrier_sem, inc=1,
                device_id=nbr, device_id_type=pl.DeviceIdType.MESH,
            )
        pl.semaphore_wait(barrier_sem, 2)

        send_lo = my_y * half
        sall[pl.ds(send_lo, half), :] = (
            x_ref[pl.ds(send_lo, half), :].astype(sall.dtype)
        )

        p1 = []
        for c in range(N_CHUNKS):
            rdma = pltpu.make_async_remote_copy(
                src_ref=sall.at[pl.ds(send_lo + c * chunk, chunk), :],
                dst_ref=rbuf.at[pl.ds(c * chunk, chunk), :],
                send_sem=p1_send.at[c],
                recv_sem=p1_recv.at[c],
                device_id=x_nbr,
                device_id_type=pl.DeviceIdType.MESH,
            )
            rdma.start()
            p1.append(rdma)

        keep_lo = (1 - my_y) * half
        sall[pl.ds(keep_lo, half), :] = (
            x_ref[pl.ds(keep_lo, half), :].astype(sall.dtype)
        )
        cp_own = pltpu.make_async_copy(
            sall, out_ref.at[pl.ds(my_x * m_per, m_per), :], dsem_own
        )
        cp_own.start()

        fwd_off = (1 - my_x) * m_per + my_y * half
        p2 = []
        downs = []
        for c in range(N_CHUNKS):
            p1[c].wait_recv()
            rdma = pltpu.make_async_remote_copy(
                src_ref=rbuf.at[pl.ds(c * chunk, chunk), :],
                dst_ref=out_ref.at[pl.ds(fwd_off + c * chunk, chunk), :],
                send_sem=p2_send.at[c],
                recv_sem=p2_recv.at[c],
                device_id=y_nbr,
                device_id_type=pl.DeviceIdType.MESH,
            )
            rdma.start()
            p2.append(rdma)
            cp = pltpu.make_async_copy(
                rbuf.at[pl.ds(c * chunk, chunk), :],
                out_ref.at[pl.ds(fwd_off + c * chunk, chunk), :],
                dsem.at[c],
            )
            cp.start()
            downs.append(cp)

        cp_own.wait()
        for c in range(N_CHUNKS):
            downs[c].wait()
            p1[c].wait_send()
            p2[c].wait()

    return pl.pallas_call(
        body,
        out_shape=jax.ShapeDtypeStruct((2 * m_per, n), jnp.bfloat16),
        in_specs=[pl.BlockSpec(memory_space=pltpu.VMEM)],
        out_specs=pl.BlockSpec(memory_space=pltpu.MemorySpace.HBM),
        scratch_shapes=[
            pltpu.VMEM((m_per, n), jnp.bfloat16),
            pltpu.VMEM((half, n), jnp.bfloat16),
            pltpu.SemaphoreType.DMA((N_CHUNKS,)),
            pltpu.SemaphoreType.DMA((N_CHUNKS,)),
            pltpu.SemaphoreType.DMA((N_CHUNKS,)),
            pltpu.SemaphoreType.DMA((N_CHUNKS,)),
            pltpu.SemaphoreType.DMA((N_CHUNKS,)),
            pltpu.SemaphoreType.DMA,
        ],
        compiler_params=pltpu.CompilerParams(collective_id=0),
    )(x)


# device time: 22117 ns/iter; 1.4837x vs baseline; 1.0351x over previous
import jax
import jax.numpy as jnp
from jax import lax
from jax.experimental import pallas as pl
from jax.experimental.pallas import tpu as pltpu

N_CHUNKS = 16


def kernel(x):
    m_per, n = x.shape
    half = m_per // 2
    chunk = half // N_CHUNKS

    def body(x_ref, out_ref, vx, sall, rbuf,
             p1_send, p1_recv, p2_send, p2_recv, dsem, dsem_own, in_sems):
        my_x = lax.axis_index("x")
        my_y = lax.axis_index("y")
        x_nbr = (1 - my_x, my_y)
        y_nbr = (my_x, 1 - my_y)

        send_lo = my_y * half
        keep_lo = (1 - my_y) * half
        cp_send = pltpu.make_async_copy(
            x_ref.at[pl.ds(send_lo, half), :],
            vx.at[pl.ds(send_lo, half), :],
            in_sems.at[0],
        )
        cp_send.start()
        cp_keep = pltpu.make_async_copy(
            x_ref.at[pl.ds(keep_lo, half), :],
            vx.at[pl.ds(keep_lo, half), :],
            in_sems.at[1],
        )
        cp_keep.start()

        barrier_sem = pltpu.get_barrier_semaphore()
        for nbr in (x_nbr, y_nbr):
            pl.semaphore_signal(
                barrier_sem, inc=1,
                device_id=nbr, device_id_type=pl.DeviceIdType.MESH,
            )
        pl.semaphore_wait(barrier_sem, 2)

        cp_send.wait()
        sall[pl.ds(send_lo, half), :] = (
            vx[pl.ds(send_lo, half), :].astype(sall.dtype)
        )

        p1 = []
        for c in range(N_CHUNKS):
            rdma = pltpu.make_async_remote_copy(
                src_ref=sall.at[pl.ds(send_lo + c * chunk, chunk), :],
                dst_ref=rbuf.at[pl.ds(c * chunk, chunk), :],
                send_sem=p1_send.at[c],
                recv_sem=p1_recv.at[c],
                device_id=x_nbr,
                device_id_type=pl.DeviceIdType.MESH,
            )
            rdma.start()
            p1.append(rdma)

        cp_keep.wait()
        sall[pl.ds(keep_lo, half), :] = (
            vx[pl.ds(keep_lo, half), :].astype(sall.dtype)
        )
        cp_own = pltpu.make_async_copy(
            sall, out_ref.at[pl.ds(my_x * m_per, m_per), :], dsem_own
        )
        cp_own.start()

        fwd_off = (1 - my_x) * m_per + my_y * half
        p2 = []
        downs = []
        for c in range(N_CHUNKS):
            p1[c].wait_recv()
            rdma = pltpu.make_async_remote_copy(
                src_ref=rbuf.at[pl.ds(c * chunk, chunk), :],
                dst_ref=out_ref.at[pl.ds(fwd_off + c * chunk, chunk), :],
                send_sem=p2_send.at[c],
                recv_sem=p2_recv.at[c],
                device_id=y_nbr,
                device_id_type=pl.DeviceIdType.MESH,
            )
            rdma.start()
            p2.append(rdma)
            cp = pltpu.make_async_copy(
                rbuf.at[pl.ds(c * chunk, chunk), :],
                out_ref.at[pl.ds(fwd_off + c * chunk, chunk), :],
                dsem.at[c],
            )
            cp.start()
            downs.append(cp)

        cp_own.wait()
        for c in range(N_CHUNKS):
            downs[c].wait()
            p1[c].wait_send()
            p2[c].wait()

    x = pltpu.with_memory_space_constraint(x, pltpu.MemorySpace.HBM)
    return pl.pallas_call(
        body,
        out_shape=jax.ShapeDtypeStruct((2 * m_per, n), jnp.bfloat16),
        in_specs=[pl.BlockSpec(memory_space=pl.ANY)],
        out_specs=pl.BlockSpec(memory_space=pl.ANY),
        scratch_shapes=[
            pltpu.VMEM((m_per, n), jnp.float32),
            pltpu.VMEM((m_per, n), jnp.bfloat16),
            pltpu.VMEM((half, n), jnp.bfloat16),
            pltpu.SemaphoreType.DMA((N_CHUNKS,)),
            pltpu.SemaphoreType.DMA((N_CHUNKS,)),
            pltpu.SemaphoreType.DMA((N_CHUNKS,)),
            pltpu.SemaphoreType.DMA((N_CHUNKS,)),
            pltpu.SemaphoreType.DMA((N_CHUNKS,)),
            pltpu.SemaphoreType.DMA,
            pltpu.SemaphoreType.DMA((2,)),
        ],
        compiler_params=pltpu.CompilerParams(collective_id=0),
    )(x)


# device time: 21596 ns/iter; 1.5195x vs baseline; 1.0241x over previous
import jax
import jax.numpy as jnp
from jax import lax
from jax.experimental import pallas as pl
from jax.experimental.pallas import tpu as pltpu

N_CHUNKS = 16
N_GROUPS = 4


def kernel(x):
    m_per, n = x.shape
    half = m_per // 2
    chunk = half // N_CHUNKS

    def body(x_ref, out_ref, vx, sall, rbuf,
             p1_send, p1_recv, p2_send, p2_recv, dsem, dsem_own, in_sems):
        my_x = lax.axis_index("x")
        my_y = lax.axis_index("y")
        x_nbr = (1 - my_x, my_y)
        y_nbr = (my_x, 1 - my_y)

        send_lo = my_y * half
        keep_lo = (1 - my_y) * half
        grows = half // N_GROUPS
        cps_send = []
        for g in range(N_GROUPS):
            cp = pltpu.make_async_copy(
                x_ref.at[pl.ds(send_lo + g * grows, grows), :],
                vx.at[pl.ds(send_lo + g * grows, grows), :],
                in_sems.at[g],
            )
            cp.start()
            cps_send.append(cp)
        cp_keep = pltpu.make_async_copy(
            x_ref.at[pl.ds(keep_lo, half), :],
            vx.at[pl.ds(keep_lo, half), :],
            in_sems.at[N_GROUPS],
        )
        cp_keep.start()

        barrier_sem = pltpu.get_barrier_semaphore()
        for nbr in (x_nbr, y_nbr):
            pl.semaphore_signal(
                barrier_sem, inc=1,
                device_id=nbr, device_id_type=pl.DeviceIdType.MESH,
            )
        pl.semaphore_wait(barrier_sem, 2)

        cpg = N_CHUNKS // N_GROUPS
        p1 = []
        for g in range(N_GROUPS):
            cps_send[g].wait()
            sall[pl.ds(send_lo + g * grows, grows), :] = (
                vx[pl.ds(send_lo + g * grows, grows), :].astype(sall.dtype)
            )
            for c in range(g * cpg, (g + 1) * cpg):
                rdma = pltpu.make_async_remote_copy(
                    src_ref=sall.at[pl.ds(send_lo + c * chunk, chunk), :],
                    dst_ref=rbuf.at[pl.ds(c * chunk, chunk), :],
                    send_sem=p1_send.at[c],
                    recv_sem=p1_recv.at[c],
                    device_id=x_nbr,
                    device_id_type=pl.DeviceIdType.MESH,
                )
                rdma.start()
                p1.append(rdma)

        cp_keep.wait()
        sall[pl.ds(keep_lo, half), :] = (
            vx[pl.ds(keep_lo, half), :].astype(sall.dtype)
        )
        cp_own = pltpu.make_async_copy(
            sall, out_ref.at[pl.ds(my_x * m_per, m_per), :], dsem_own
        )
        cp_own.start()

        fwd_off = (1 - my_x) * m_per + my_y * half
        p2 = []
        downs = []
        for c in range(N_CHUNKS):
            p1[c].wait_recv()
            rdma = pltpu.make_async_remote_copy(
                src_ref=rbuf.at[pl.ds(c * chunk, chunk), :],
                dst_ref=out_ref.at[pl.ds(fwd_off + c * chunk, chunk), :],
                send_sem=p2_send.at[c],
                recv_sem=p2_recv.at[c],
                device_id=y_nbr,
                device_id_type=pl.DeviceIdType.MESH,
            )
            rdma.start()
            p2.append(rdma)
            cp = pltpu.make_async_copy(
                rbuf.at[pl.ds(c * chunk, chunk), :],
                out_ref.at[pl.ds(fwd_off + c * chunk, chunk), :],
                dsem.at[c],
            )
            cp.start()
            downs.append(cp)

        cp_own.wait()
        for c in range(N_CHUNKS):
            downs[c].wait()
            p1[c].wait_send()
            p2[c].wait()

    x = pltpu.with_memory_space_constraint(x, pltpu.MemorySpace.HBM)
    return pl.pallas_call(
        body,
        out_shape=jax.ShapeDtypeStruct((2 * m_per, n), jnp.bfloat16),
        in_specs=[pl.BlockSpec(memory_space=pl.ANY)],
        out_specs=pl.BlockSpec(memory_space=pl.ANY),
        scratch_shapes=[
            pltpu.VMEM((m_per, n), jnp.float32),
            pltpu.VMEM((m_per, n), jnp.bfloat16),
            pltpu.VMEM((half, n), jnp.bfloat16),
            pltpu.SemaphoreType.DMA((N_CHUNKS,)),
            pltpu.SemaphoreType.DMA((N_CHUNKS,)),
            pltpu.SemaphoreType.DMA((N_CHUNKS,)),
            pltpu.SemaphoreType.DMA((N_CHUNKS,)),
            pltpu.SemaphoreType.DMA((N_CHUNKS,)),
            pltpu.SemaphoreType.DMA,
            pltpu.SemaphoreType.DMA((N_GROUPS + 1,)),
        ],
        compiler_params=pltpu.CompilerParams(collective_id=0),
    )(x)


# device time: 21521 ns/iter; 1.5248x vs baseline; 1.0035x over previous
import jax
import jax.numpy as jnp
from jax import lax
from jax.experimental import pallas as pl
from jax.experimental.pallas import tpu as pltpu

N_GROUPS = 8


def _chunk_layout(half):
    big = 64
    return [(o, big) for o in range(0, half, big)]


def kernel(x):
    m_per, n = x.shape
    half = m_per // 2
    chunks = _chunk_layout(half)
    n_chunks = len(chunks)

    def body(x_ref, out_ref, vx, sall, rbuf,
             p1_send, p1_recv, p2_send, p2_recv, dsem, dsem_own, in_sems):
        my_x = lax.axis_index("x")
        my_y = lax.axis_index("y")
        x_nbr = (1 - my_x, my_y)
        y_nbr = (my_x, 1 - my_y)

        send_lo = my_y * half
        keep_lo = (1 - my_y) * half
        grows = half // N_GROUPS
        cps_send = []
        for g in range(N_GROUPS):
            cp = pltpu.make_async_copy(
                x_ref.at[pl.ds(send_lo + g * grows, grows), :],
                vx.at[pl.ds(send_lo + g * grows, grows), :],
                in_sems.at[g],
            )
            cp.start()
            cps_send.append(cp)
        cp_keep = pltpu.make_async_copy(
            x_ref.at[pl.ds(keep_lo, half), :],
            vx.at[pl.ds(keep_lo, half), :],
            in_sems.at[N_GROUPS],
        )
        cp_keep.start()

        barrier_sem = pltpu.get_barrier_semaphore()
        for nbr in (x_nbr, y_nbr):
            pl.semaphore_signal(
                barrier_sem, inc=1,
                device_id=nbr, device_id_type=pl.DeviceIdType.MESH,
            )
        pl.semaphore_wait(barrier_sem, 2)

        p1 = []
        for g in range(N_GROUPS):
            cps_send[g].wait()
            sall[pl.ds(send_lo + g * grows, grows), :] = (
                vx[pl.ds(send_lo + g * grows, grows), :].astype(sall.dtype)
            )
            for c, (off, sz) in enumerate(chunks):
                if not (g * grows <= off < (g + 1) * grows):
                    continue
                rdma = pltpu.make_async_remote_copy(
                    src_ref=sall.at[pl.ds(send_lo + off, sz), :],
                    dst_ref=rbuf.at[pl.ds(off, sz), :],
                    send_sem=p1_send.at[c],
                    recv_sem=p1_recv.at[c],
                    device_id=x_nbr,
                    device_id_type=pl.DeviceIdType.MESH,
                )
                rdma.start()
                p1.append(rdma)

        cp_keep.wait()
        sall[pl.ds(keep_lo, half), :] = (
            vx[pl.ds(keep_lo, half), :].astype(sall.dtype)
        )
        cp_own = pltpu.make_async_copy(
            sall, out_ref.at[pl.ds(my_x * m_per, m_per), :], dsem_own
        )
        cp_own.start()

        fwd_off = (1 - my_x) * m_per + my_y * half
        p2 = []
        downs = []
        for c, (off, sz) in enumerate(chunks):
            p1[c].wait_recv()
            rdma = pltpu.make_async_remote_copy(
                src_ref=rbuf.at[pl.ds(off, sz), :],
                dst_ref=out_ref.at[pl.ds(fwd_off + off, sz), :],
                send_sem=p2_send.at[c],
                recv_sem=p2_recv.at[c],
                device_id=y_nbr,
                device_id_type=pl.DeviceIdType.MESH,
            )
            rdma.start()
            p2.append(rdma)
            cp = pltpu.make_async_copy(
                rbuf.at[pl.ds(off, sz), :],
                out_ref.at[pl.ds(fwd_off + off, sz), :],
                dsem.at[c],
            )
            cp.start()
            downs.append(cp)

        cp_own.wait()
        for c in range(n_chunks):
            downs[c].wait()
            p1[c].wait_send()
            p2[c].wait()

    x = pltpu.with_memory_space_constraint(x, pltpu.MemorySpace.HBM)
    return pl.pallas_call(
        body,
        out_shape=jax.ShapeDtypeStruct((2 * m_per, n), jnp.bfloat16),
        in_specs=[pl.BlockSpec(memory_space=pl.ANY)],
        out_specs=pl.BlockSpec(memory_space=pl.ANY),
        scratch_shapes=[
            pltpu.VMEM((m_per, n), jnp.float32),
            pltpu.VMEM((m_per, n), jnp.bfloat16),
            pltpu.VMEM((half, n), jnp.bfloat16),
            pltpu.SemaphoreType.DMA((n_chunks,)),
            pltpu.SemaphoreType.DMA((n_chunks,)),
            pltpu.SemaphoreType.DMA((n_chunks,)),
            pltpu.SemaphoreType.DMA((n_chunks,)),
            pltpu.SemaphoreType.DMA((n_chunks,)),
            pltpu.SemaphoreType.DMA,
            pltpu.SemaphoreType.DMA((N_GROUPS + 1,)),
        ],
        compiler_params=pltpu.CompilerParams(collective_id=0),
    )(x)
